# baseline (speedup 1.0000x reference)
import jax
import jax.numpy as jnp
from jax import lax
from jax.experimental import pallas as pl
from jax.experimental.pallas import tpu as pltpu
from jax.experimental.pallas import tpu_sc as plsc

_N = 10000
_F = 128
_E = 320000
_NP = 10240
_W = 128
_LANES = 16
_NTILES = 16
_NSC = 2
_EPAD = 327680
_NWIN = _EPAD // (_NSC * _NTILES * _W)
_HR = _NP // _F


def _sc_pass(with_deg):
    mesh = plsc.VectorSubcoreMesh(core_axis_name="c", subcore_axis_name="s")
    out_type = [jax.ShapeDtypeStruct((_NSC, _NP, _F), jnp.float32)]
    scratch = [
        pltpu.VMEM((2, _W), jnp.int32),
        pltpu.VMEM((2, _W), jnp.int32),
        pltpu.VMEM((_W,), jnp.int32),
        pltpu.VMEM((_W,), jnp.int32),
        pltpu.VMEM((_W, _F), jnp.float32),
        pltpu.VMEM((_W, _F), jnp.float32),
        pltpu.VMEM_SHARED((_NP, _F), jnp.float32),
        pltpu.SemaphoreType.DMA,
        pltpu.SemaphoreType.DMA,
        pltpu.SemaphoreType.DMA,
        pltpu.SemaphoreType.DMA,
        pltpu.SemaphoreType.DMA,
        pltpu.SemaphoreType.DMA,
    ]
    if with_deg:
        out_type.append(
            jax.ShapeDtypeStruct((_NSC * _NTILES, _HR, _F), jnp.int32))
        scratch.append(pltpu.VMEM((_HR, _F), jnp.int32))

    def body(*args):
        if with_deg:
            (v_hbm, idx_hbm, zf_hbm, zh_hbm, s_out, d_out,
             idx0, idx1, colp0, colp1, xb0, xb1, agg,
             isem0, isem1, gsem0, gsem1, ssem0, ssem1, hist_v) = args
        else:
            (v_hbm, idx_hbm, zf_hbm, s_out,
             idx0, idx1, colp0, colp1, xb0, xb1, agg,
             isem0, isem1, gsem0, gsem1, ssem0, ssem1) = args
        idx = (idx0, idx1)
        colp = (colp0, colp1)
        xb = (xb0, xb1)
        isem = (isem0, isem1)
        gsem = (gsem0, gsem1)
        ssem = (ssem0, ssem1)
        c = lax.axis_index("c")
        s = lax.axis_index("s")
        zrows = _NP // _NTILES
        pltpu.sync_copy(zf_hbm.at[pl.ds(s * zrows, zrows)],
                        agg.at[pl.ds(s * zrows, zrows)])
        if with_deg:
            pltpu.sync_copy(zh_hbm, hist_v)
        plsc.subcore_barrier()

        lane = lax.iota(jnp.int32, 16)
        wbase = (c * _NTILES + s) * _NWIN

        def compute_colp(b):
            rv, cv = idx[b].at[0], idx[b].at[1]
            for i in range(_W // 16):
                r = rv[pl.ds(i * 16, 16)]
                cc = cv[pl.ds(i * 16, 16)]
                colp[b][pl.ds(i * 16, 16)] = jnp.where(r == cc, _N + lane, cc)

        def deg_update(b):
            rv, cv = idx[b].at[0], idx[b].at[1]
            for i in range(_W // 16):
                r = rv[pl.ds(i * 16, 16)]
                cc = cv[pl.ds(i * 16, 16)]
                cnt, last = plsc.scan_count(cc, r != cc)
                plsc.addupdate_scatter(
                    hist_v, [cc >> 7, cc & 127], cnt, mask=last)

        def start_gather(b):
            pltpu.async_copy(v_hbm.at[idx[b].at[0]], xb[b], gsem[b])

        def wait_gather(b):
            pltpu.make_async_copy(v_hbm.at[idx[b].at[0]], xb[b],
                                  gsem[b]).wait()

        def start_scatter(b):
            pltpu.async_copy(xb[b], agg.at[colp[b]], ssem[b], add=True)

        def wait_scatter(b):
            pltpu.make_async_copy(xb[b], agg.at[colp[b]], ssem[b]).wait()

        pltpu.sync_copy(idx_hbm.at[wbase], idx0)
        compute_colp(0)
        start_gather(0)

        def pair(k, carry):
            for b in (0, 1):
                g = 2 * k + b
                o = 1 - b
                pltpu.async_copy(idx_hbm.at[wbase + g + 1], idx[o], isem[o])
                if with_deg:
                    deg_update(b)
                wait_gather(b)
                start_scatter(b)
                if b == 0:

                    @pl.when(k > 0)
                    def _():
                        wait_scatter(o)
                else:
                    wait_scatter(o)
                pltpu.make_async_copy(idx_hbm.at[wbase + g + 1], idx[o],
                                      isem[o]).wait()
                compute_colp(o)
                start_gather(o)
            return carry

        lax.fori_loop(0, _NWIN // 2, pair, 0)
        wait_scatter(1)
        wait_gather(0)
        plsc.subcore_barrier()
        orows = _NP // _NTILES
        pltpu.sync_copy(agg.at[pl.ds(s * orows, orows)],
                        s_out.at[c, pl.ds(s * orows, orows)])
        if with_deg:
            pltpu.sync_copy(hist_v, d_out.at[c * _NTILES + s])

    params = (pltpu.CompilerParams(needs_layout_passes=False)
              if with_deg else None)
    return pl.kernel(body, out_type=out_type, mesh=mesh,
                     scratch_types=scratch, compiler_params=params)


_R = 1000
_RP = 1024


def _lrelu(h):
    return jnp.where(h >= 0, h, 0.01 * h)


def _prep_body(d_ref, o_ref):
    deg = jnp.sum(d_ref[...], axis=0).astype(jnp.float32) + 1.0
    dinv = 1.0 / deg
    ey = (lax.broadcasted_iota(jnp.int32, (_F, _F), 0)
          == lax.broadcasted_iota(jnp.int32, (_F, _F), 1)).astype(jnp.float32)
    ones = jnp.ones((_F, _F), jnp.float32)
    for r in range(_RP // _F):
        dg = ey * dinv[r:r + 1, :]
        o_ref[pl.ds(r * _F, _F), :] = jnp.dot(
            dg, ones, preferred_element_type=jnp.float32)


def _prep(dp):
    return pl.pallas_call(
        _prep_body,
        grid=(_NP // _RP,),
        in_specs=[pl.BlockSpec((_NSC * _NTILES, _RP // _F, _F),
                               lambda i: (0, i, 0))],
        out_specs=pl.BlockSpec((_RP, _F), lambda i: (i, 0)),
        out_shape=jax.ShapeDtypeStruct((_NP, _F), jnp.float32),
    )(dp)


def _root_body(v_ref, w_ref, o_ref):
    o_ref[...] = jnp.dot(v_ref[...], w_ref[...],
                         preferred_element_type=jnp.float32)


def _root(v, w):
    fin, fout = w.shape
    return pl.pallas_call(
        _root_body,
        grid=(_N // _R,),
        in_specs=[pl.BlockSpec((_R, fin), lambda i: (i, 0)),
                  _full((fin, fout))],
        out_specs=pl.BlockSpec((_R, fout), lambda i: (i, 0)),
        out_shape=jax.ShapeDtypeStruct((_N, fout), jnp.float32),
    )(v, w)


def _tc1_body(x_ref, s_ref, dv_ref, wo_ref, b_ref, r1_ref, w2_ref,
              h1_ref, y2_ref):
    xb = x_ref[...]
    agg = (s_ref[0] + s_ref[1] + xb) * dv_ref[...]
    h = (jnp.dot(agg, wo_ref[...], preferred_element_type=jnp.float32)
         + b_ref[...] + r1_ref[...])
    h1 = _lrelu(h)
    h1_ref[...] = h1
    y2_ref[...] = jnp.dot(h1, w2_ref[...], preferred_element_type=jnp.float32)


def _tc2_body(y_ref, s_ref, dv_ref, r2_ref, b2_ref, w3_ref, b3_ref,
              w4_ref, b4_ref, w5_ref, b5_ref, o_ref):
    yb = y_ref[...]
    h2 = _lrelu((s_ref[0] + s_ref[1] + yb) * dv_ref[...] + b2_ref[...]
                + r2_ref[...])
    h3 = _lrelu(jnp.dot(h2, w3_ref[...],
                        preferred_element_type=jnp.float32) + b3_ref[...])
    h4 = _lrelu(jnp.dot(h3, w4_ref[...],
                        preferred_element_type=jnp.float32) + b4_ref[...])
    o_ref[...] = jnp.dot(h4, w5_ref[...],
                         preferred_element_type=jnp.float32) + b5_ref[...]


def _full(shape):
    return pl.BlockSpec(shape, lambda i: tuple(0 for _ in shape))


def _tc1(x, s1, dv, w1_out, b1, r1, w2_out):
    return pl.pallas_call(
        _tc1_body,
        grid=(_N // _R,),
        in_specs=[
            pl.BlockSpec((_R, _F), lambda i: (i, 0)),
            pl.BlockSpec((_NSC, _R, _F), lambda i: (0, i, 0)),
            pl.BlockSpec((_R, _F), lambda i: (i, 0)),
            _full((_F, 256)),
            _full((1, 256)),
            pl.BlockSpec((_R, 256), lambda i: (i, 0)),
            _full((256, _F)),
        ],
        out_specs=[
            pl.BlockSpec((_R, 256), lambda i: (i, 0)),
            pl.BlockSpec((_R, _F), lambda i: (i, 0)),
        ],
        out_shape=[
            jax.ShapeDtypeStruct((_N, 256), jnp.float32),
            jax.ShapeDtypeStruct((_N, _F), jnp.float32),
        ],
    )(x, s1, dv, w1_out, b1, r1, w2_out)


def _tc2(y2, s2, dv, r2, b2, w3, b3, w4, b4, w5, b5):
    return pl.pallas_call(
        _tc2_body,
        grid=(_N // _R,),
        in_specs=[
            pl.BlockSpec((_R, _F), lambda i: (i, 0)),
            pl.BlockSpec((_NSC, _R, _F), lambda i: (0, i, 0)),
            pl.BlockSpec((_R, _F), lambda i: (i, 0)),
            pl.BlockSpec((_R, _F), lambda i: (i, 0)),
            _full((1, _F)),
            _full((_F, 256)),
            _full((1, 256)),
            _full((256, _F)),
            _full((1, _F)),
            _full((_F, 64)),
            _full((1, 64)),
        ],
        out_specs=pl.BlockSpec((_R, 64), lambda i: (i, 0)),
        out_shape=jax.ShapeDtypeStruct((_N, 64), jnp.float32),
    )(y2, s2, dv, r2, b2, w3, b3, w4, b4, w5, b5)


def kernel(x, edge_index, w1_out, b1_out, w1_root, w2_out, b2_out, w2_root,
           w3, b3, w4, b4, w5, b5):
    nwin_total = _EPAD // _W + 1
    pad = (jnp.arange(_E, _EPAD + _W, dtype=jnp.int32) % _N)
    rowp = jnp.concatenate([edge_index[0], pad]).reshape(nwin_total, 1, _W)
    colp = jnp.concatenate([edge_index[1], pad]).reshape(nwin_total, 1, _W)
    idxarr = jnp.concatenate([rowp, colp], axis=1)
    zf = jnp.zeros((_NP, _F), jnp.float32)
    zh = jnp.zeros((_HR, _F), jnp.int32)

    s1, dp = _sc_pass(True)(x, idxarr, zf, zh)
    r1 = _root(x, w1_root)
    dv = _prep(dp)
    h1, y2 = _tc1(x, s1, dv, w1_out, b1_out.reshape(1, -1), r1, w2_out)
    (s2,) = _sc_pass(False)(y2, idxarr, zf)
    r2 = _root(h1, w2_root)
    return _tc2(y2, s2, dv, r2, b2_out.reshape(1, -1),
                w3, b3.reshape(1, -1), w4, b4.reshape(1, -1),
                w5, b5.reshape(1, -1))


# speedup vs baseline: 1.0036x; 1.0036x over previous
import jax
import jax.numpy as jnp
from jax import lax
from jax.experimental import pallas as pl
from jax.experimental.pallas import tpu as pltpu
from jax.experimental.pallas import tpu_sc as plsc

_N = 10000
_F = 128
_E = 320000
_NP = 10240
_W = 128
_LANES = 16
_NTILES = 16
_NSC = 2
_EPAD = 327680
_NWIN = _EPAD // (_NSC * _NTILES * _W)
_HR = _NP // _F


def _sc_pass(with_deg):
    mesh = plsc.VectorSubcoreMesh(core_axis_name="c", subcore_axis_name="s")
    out_type = [jax.ShapeDtypeStruct((_NSC, _NP, _F), jnp.float32)]
    scratch = [
        pltpu.VMEM((2, _W), jnp.int32),
        pltpu.VMEM((2, _W), jnp.int32),
        pltpu.VMEM((_W,), jnp.int32),
        pltpu.VMEM((_W,), jnp.int32),
        pltpu.VMEM((_W, _F), jnp.float32),
        pltpu.VMEM((_W, _F), jnp.float32),
        pltpu.VMEM_SHARED((_NP, _F), jnp.float32),
        pltpu.SemaphoreType.DMA,
        pltpu.SemaphoreType.DMA,
        pltpu.SemaphoreType.DMA,
        pltpu.SemaphoreType.DMA,
        pltpu.SemaphoreType.DMA,
        pltpu.SemaphoreType.DMA,
    ]
    if with_deg:
        out_type.append(
            jax.ShapeDtypeStruct((_NSC * _NTILES, _HR, _F), jnp.int32))
        scratch.append(pltpu.VMEM((_HR, _F), jnp.int32))

    def body(*args):
        if with_deg:
            (v_hbm, idx_hbm, zf_hbm, zh_hbm, s_out, d_out,
             idx0, idx1, colp0, colp1, xb0, xb1, agg,
             isem0, isem1, gsem0, gsem1, ssem0, ssem1, hist_v) = args
        else:
            (v_hbm, idx_hbm, zf_hbm, s_out,
             idx0, idx1, colp0, colp1, xb0, xb1, agg,
             isem0, isem1, gsem0, gsem1, ssem0, ssem1) = args
        idx = (idx0, idx1)
        colp = (colp0, colp1)
        xb = (xb0, xb1)
        isem = (isem0, isem1)
        gsem = (gsem0, gsem1)
        ssem = (ssem0, ssem1)
        c = lax.axis_index("c")
        s = lax.axis_index("s")
        zrows = _NP // _NTILES
        pltpu.sync_copy(zf_hbm.at[pl.ds(s * zrows, zrows)],
                        agg.at[pl.ds(s * zrows, zrows)])
        if with_deg:
            pltpu.sync_copy(zh_hbm, hist_v)
        plsc.subcore_barrier()

        lane = lax.iota(jnp.int32, 16)
        wbase = (c * _NTILES + s) * _NWIN

        def compute_colp(b):
            rv, cv = idx[b].at[0], idx[b].at[1]
            for i in range(_W // 16):
                r = rv[pl.ds(i * 16, 16)]
                cc = cv[pl.ds(i * 16, 16)]
                colp[b][pl.ds(i * 16, 16)] = jnp.where(r == cc, _N + lane, cc)

        def deg_update(b):
            rv, cv = idx[b].at[0], idx[b].at[1]
            for i in range(_W // 16):
                r = rv[pl.ds(i * 16, 16)]
                cc = cv[pl.ds(i * 16, 16)]
                cnt, last = plsc.scan_count(cc, r != cc)
                plsc.addupdate_scatter(
                    hist_v, [cc >> 7, cc & 127], cnt, mask=last)

        def start_gather(b):
            pltpu.async_copy(v_hbm.at[idx[b].at[0]], xb[b], gsem[b])

        def wait_gather(b):
            pltpu.make_async_copy(v_hbm.at[idx[b].at[0]], xb[b],
                                  gsem[b]).wait()

        def start_scatter(b):
            pltpu.async_copy(xb[b], agg.at[colp[b]], ssem[b], add=True)

        def wait_scatter(b):
            pltpu.make_async_copy(xb[b], agg.at[colp[b]], ssem[b]).wait()

        pltpu.sync_copy(idx_hbm.at[wbase], idx0)
        compute_colp(0)
        start_gather(0)

        def pair(k, carry):
            for b in (0, 1):
                g = 2 * k + b
                o = 1 - b
                pltpu.async_copy(idx_hbm.at[wbase + g + 1], idx[o], isem[o])
                if with_deg:
                    deg_update(b)
                wait_gather(b)
                start_scatter(b)
                if b == 0:

                    @pl.when(k > 0)
                    def _():
                        wait_scatter(o)
                else:
                    wait_scatter(o)
                pltpu.make_async_copy(idx_hbm.at[wbase + g + 1], idx[o],
                                      isem[o]).wait()
                compute_colp(o)
                start_gather(o)
            return carry

        lax.fori_loop(0, _NWIN // 2, pair, 0)
        wait_scatter(1)
        wait_gather(0)
        plsc.subcore_barrier()
        orows = _NP // _NTILES
        pltpu.sync_copy(agg.at[pl.ds(s * orows, orows)],
                        s_out.at[c, pl.ds(s * orows, orows)])
        if with_deg:
            pltpu.sync_copy(hist_v, d_out.at[c * _NTILES + s])

    params = (pltpu.CompilerParams(needs_layout_passes=False)
              if with_deg else None)
    cost = pl.CostEstimate(flops=0, transcendentals=0,
                           bytes_accessed=340_000_000)
    return pl.kernel(body, out_type=out_type, mesh=mesh,
                     scratch_types=scratch, compiler_params=params,
                     cost_estimate=cost)


_R = 1000
_RP = 1024


def _lrelu(h):
    return jnp.where(h >= 0, h, 0.01 * h)


def _prep_body(d_ref, o_ref):
    deg = jnp.sum(d_ref[...], axis=0).astype(jnp.float32) + 1.0
    dinv = 1.0 / deg
    ey = (lax.broadcasted_iota(jnp.int32, (_F, _F), 0)
          == lax.broadcasted_iota(jnp.int32, (_F, _F), 1)).astype(jnp.float32)
    ones = jnp.ones((_F, _F), jnp.float32)
    for r in range(_RP // _F):
        dg = ey * dinv[r:r + 1, :]
        o_ref[pl.ds(r * _F, _F), :] = jnp.dot(
            dg, ones, preferred_element_type=jnp.float32)


def _prep(dp):
    return pl.pallas_call(
        _prep_body,
        grid=(_NP // _RP,),
        in_specs=[pl.BlockSpec((_NSC * _NTILES, _RP // _F, _F),
                               lambda i: (0, i, 0))],
        out_specs=pl.BlockSpec((_RP, _F), lambda i: (i, 0)),
        out_shape=jax.ShapeDtypeStruct((_NP, _F), jnp.float32),
    )(dp)


def _root_body(v_ref, w_ref, o_ref):
    o_ref[...] = jnp.dot(v_ref[...], w_ref[...],
                         preferred_element_type=jnp.float32)


def _root(v, w):
    fin, fout = w.shape
    return pl.pallas_call(
        _root_body,
        grid=(_N // _R,),
        in_specs=[pl.BlockSpec((_R, fin), lambda i: (i, 0)),
                  _full((fin, fout))],
        out_specs=pl.BlockSpec((_R, fout), lambda i: (i, 0)),
        out_shape=jax.ShapeDtypeStruct((_N, fout), jnp.float32),
    )(v, w)


def _tc1_body(x_ref, s_ref, dv_ref, wo_ref, b_ref, r1_ref, w2_ref,
              h1_ref, y2_ref):
    xb = x_ref[...]
    agg = (s_ref[0] + s_ref[1] + xb) * dv_ref[...]
    h = (jnp.dot(agg, wo_ref[...], preferred_element_type=jnp.float32)
         + b_ref[...] + r1_ref[...])
    h1 = _lrelu(h)
    h1_ref[...] = h1
    y2_ref[...] = jnp.dot(h1, w2_ref[...], preferred_element_type=jnp.float32)


def _tc2_body(y_ref, s_ref, dv_ref, r2_ref, b2_ref, w3_ref, b3_ref,
              w4_ref, b4_ref, w5_ref, b5_ref, o_ref):
    yb = y_ref[...]
    h2 = _lrelu((s_ref[0] + s_ref[1] + yb) * dv_ref[...] + b2_ref[...]
                + r2_ref[...])
    h3 = _lrelu(jnp.dot(h2, w3_ref[...],
                        preferred_element_type=jnp.float32) + b3_ref[...])
    h4 = _lrelu(jnp.dot(h3, w4_ref[...],
                        preferred_element_type=jnp.float32) + b4_ref[...])
    o_ref[...] = jnp.dot(h4, w5_ref[...],
                         preferred_element_type=jnp.float32) + b5_ref[...]


def _full(shape):
    return pl.BlockSpec(shape, lambda i: tuple(0 for _ in shape))


def _tc1(x, s1, dv, w1_out, b1, r1, w2_out):
    return pl.pallas_call(
        _tc1_body,
        grid=(_N // _R,),
        in_specs=[
            pl.BlockSpec((_R, _F), lambda i: (i, 0)),
            pl.BlockSpec((_NSC, _R, _F), lambda i: (0, i, 0)),
            pl.BlockSpec((_R, _F), lambda i: (i, 0)),
            _full((_F, 256)),
            _full((1, 256)),
            pl.BlockSpec((_R, 256), lambda i: (i, 0)),
            _full((256, _F)),
        ],
        out_specs=[
            pl.BlockSpec((_R, 256), lambda i: (i, 0)),
            pl.BlockSpec((_R, _F), lambda i: (i, 0)),
        ],
        out_shape=[
            jax.ShapeDtypeStruct((_N, 256), jnp.float32),
            jax.ShapeDtypeStruct((_N, _F), jnp.float32),
        ],
    )(x, s1, dv, w1_out, b1, r1, w2_out)


def _tc2(y2, s2, dv, r2, b2, w3, b3, w4, b4, w5, b5):
    return pl.pallas_call(
        _tc2_body,
        grid=(_N // _R,),
        in_specs=[
            pl.BlockSpec((_R, _F), lambda i: (i, 0)),
            pl.BlockSpec((_NSC, _R, _F), lambda i: (0, i, 0)),
            pl.BlockSpec((_R, _F), lambda i: (i, 0)),
            pl.BlockSpec((_R, _F), lambda i: (i, 0)),
            _full((1, _F)),
            _full((_F, 256)),
            _full((1, 256)),
            _full((256, _F)),
            _full((1, _F)),
            _full((_F, 64)),
            _full((1, 64)),
        ],
        out_specs=pl.BlockSpec((_R, 64), lambda i: (i, 0)),
        out_shape=jax.ShapeDtypeStruct((_N, 64), jnp.float32),
    )(y2, s2, dv, r2, b2, w3, b3, w4, b4, w5, b5)


def kernel(x, edge_index, w1_out, b1_out, w1_root, w2_out, b2_out, w2_root,
           w3, b3, w4, b4, w5, b5):
    nwin_total = _EPAD // _W + 1
    pad = (jnp.arange(_E, _EPAD + _W, dtype=jnp.int32) % _N)
    rowp = jnp.concatenate([edge_index[0], pad]).reshape(nwin_total, 1, _W)
    colp = jnp.concatenate([edge_index[1], pad]).reshape(nwin_total, 1, _W)
    idxarr = jnp.concatenate([rowp, colp], axis=1)
    zf = jnp.zeros((_NP, _F), jnp.float32)
    zh = jnp.zeros((_HR, _F), jnp.int32)

    s1, dp = _sc_pass(True)(x, idxarr, zf, zh)
    r1 = _root(x, w1_root)
    dv = _prep(dp)
    h1, y2 = _tc1(x, s1, dv, w1_out, b1_out.reshape(1, -1), r1, w2_out)
    (s2,) = _sc_pass(False)(y2, idxarr, zf)
    r2 = _root(h1, w2_root)
    return _tc2(y2, s2, dv, r2, b2_out.reshape(1, -1),
                w3, b3.reshape(1, -1), w4, b4.reshape(1, -1),
                w5, b5.reshape(1, -1))


# speedup vs baseline: 1.0161x; 1.0124x over previous
import jax
import jax.numpy as jnp
from jax import lax
from jax.experimental import pallas as pl
from jax.experimental.pallas import tpu as pltpu
from jax.experimental.pallas import tpu_sc as plsc

_N = 10000
_F = 128
_E = 320000
_NP = 10240
_W = 128
_LANES = 16
_NTILES = 16
_NSC = 2
_EPAD = 327680
_NWIN = _EPAD // (_NSC * _NTILES * _W)
_HR = _NP // _F


def _sc_pass(with_deg):
    mesh = plsc.VectorSubcoreMesh(core_axis_name="c", subcore_axis_name="s")
    out_type = [jax.ShapeDtypeStruct((_NSC, _NP, _F), jnp.float32)]
    scratch = [
        pltpu.VMEM((2, _W), jnp.int32),
        pltpu.VMEM((2, _W), jnp.int32),
        pltpu.VMEM((_W,), jnp.int32),
        pltpu.VMEM((_W,), jnp.int32),
        pltpu.VMEM((_W, _F), jnp.float32),
        pltpu.VMEM((_W, _F), jnp.float32),
        pltpu.VMEM_SHARED((_NP, _F), jnp.float32),
        pltpu.SemaphoreType.DMA,
        pltpu.SemaphoreType.DMA,
        pltpu.SemaphoreType.DMA,
        pltpu.SemaphoreType.DMA,
        pltpu.SemaphoreType.DMA,
        pltpu.SemaphoreType.DMA,
    ]
    if with_deg:
        out_type.append(
            jax.ShapeDtypeStruct((_NSC * _NTILES, _HR, _F), jnp.int32))
        scratch.append(pltpu.VMEM((_HR, _F), jnp.int32))

    def body(*args):
        if with_deg:
            (v_hbm, idx_hbm, zf_hbm, zh_hbm, s_out, d_out,
             idx0, idx1, colp0, colp1, xb0, xb1, agg,
             isem0, isem1, gsem0, gsem1, ssem0, ssem1, hist_v) = args
        else:
            (v_hbm, idx_hbm, zf_hbm, s_out,
             idx0, idx1, colp0, colp1, xb0, xb1, agg,
             isem0, isem1, gsem0, gsem1, ssem0, ssem1) = args
        idx = (idx0, idx1)
        colp = (colp0, colp1)
        xb = (xb0, xb1)
        isem = (isem0, isem1)
        gsem = (gsem0, gsem1)
        ssem = (ssem0, ssem1)
        c = lax.axis_index("c")
        s = lax.axis_index("s")
        zrows = _NP // _NTILES
        pltpu.sync_copy(zf_hbm.at[pl.ds(s * zrows, zrows)],
                        agg.at[pl.ds(s * zrows, zrows)])
        if with_deg:
            pltpu.sync_copy(zh_hbm, hist_v)
        plsc.subcore_barrier()

        lane = lax.iota(jnp.int32, 16)
        wbase = (c * _NTILES + s) * _NWIN

        def compute_colp(b):
            rv, cv = idx[b].at[0], idx[b].at[1]
            for i in range(_W // 16):
                r = rv[pl.ds(i * 16, 16)]
                cc = cv[pl.ds(i * 16, 16)]
                colp[b][pl.ds(i * 16, 16)] = jnp.where(r == cc, _N + lane, cc)

        def deg_update(b):
            rv, cv = idx[b].at[0], idx[b].at[1]
            for i in range(_W // 16):
                r = rv[pl.ds(i * 16, 16)]
                cc = cv[pl.ds(i * 16, 16)]
                cnt, last = plsc.scan_count(cc, r != cc)
                plsc.addupdate_scatter(
                    hist_v, [cc >> 7, cc & 127], cnt, mask=last)

        def start_gather(b):
            pltpu.async_copy(v_hbm.at[idx[b].at[0]], xb[b], gsem[b])

        def wait_gather(b):
            pltpu.make_async_copy(v_hbm.at[idx[b].at[0]], xb[b],
                                  gsem[b]).wait()

        def start_scatter(b):
            pltpu.async_copy(xb[b], agg.at[colp[b]], ssem[b], add=True)

        def wait_scatter(b):
            pltpu.make_async_copy(xb[b], agg.at[colp[b]], ssem[b]).wait()

        pltpu.sync_copy(idx_hbm.at[wbase], idx0)
        compute_colp(0)
        start_gather(0)

        def pair(k, carry):
            for b in (0, 1):
                g = 2 * k + b
                o = 1 - b
                pltpu.async_copy(idx_hbm.at[wbase + g + 1], idx[o], isem[o])
                if with_deg:
                    deg_update(b)
                wait_gather(b)
                start_scatter(b)
                if b == 0:

                    @pl.when(k > 0)
                    def _():
                        wait_scatter(o)
                else:
                    wait_scatter(o)
                pltpu.make_async_copy(idx_hbm.at[wbase + g + 1], idx[o],
                                      isem[o]).wait()
                compute_colp(o)
                start_gather(o)
            return carry

        lax.fori_loop(0, _NWIN // 2, pair, 0)
        wait_scatter(1)
        wait_gather(0)
        plsc.subcore_barrier()
        orows = _NP // _NTILES
        pltpu.sync_copy(agg.at[pl.ds(s * orows, orows)],
                        s_out.at[c, pl.ds(s * orows, orows)])
        if with_deg:
            pltpu.sync_copy(hist_v, d_out.at[c * _NTILES + s])

    params = (pltpu.CompilerParams(needs_layout_passes=False)
              if with_deg else None)
    cost = pl.CostEstimate(flops=0, transcendentals=0,
                           bytes_accessed=340_000_000)
    return pl.kernel(body, out_type=out_type, mesh=mesh,
                     scratch_types=scratch, compiler_params=params,
                     cost_estimate=cost)


_RP = 1024


def _lrelu(h):
    return jnp.where(h >= 0, h, 0.01 * h)


def _dv_block(d_ref):
    deg = jnp.sum(d_ref[...], axis=0).astype(jnp.float32) + 1.0
    dinv = 1.0 / deg
    ey = (lax.broadcasted_iota(jnp.int32, (_F, _F), 0)
          == lax.broadcasted_iota(jnp.int32, (_F, _F), 1)).astype(jnp.float32)
    ones = jnp.ones((_F, _F), jnp.float32)
    rows = [jnp.dot(ey * dinv[r:r + 1, :], ones,
                    preferred_element_type=jnp.float32)
            for r in range(_RP // _F)]
    return jnp.concatenate(rows, axis=0)


def _tc1_body(x_ref, s_ref, d_ref, wo_ref, b_ref, wr_ref, w2_ref,
              h1_ref, y2_ref):
    xb = x_ref[...]
    agg = (s_ref[0] + s_ref[1] + xb) * _dv_block(d_ref)
    h = (jnp.dot(agg, wo_ref[...], preferred_element_type=jnp.float32)
         + b_ref[...]
         + jnp.dot(xb, wr_ref[...], preferred_element_type=jnp.float32))
    h1 = _lrelu(h)
    h1_ref[...] = h1
    y2_ref[...] = jnp.dot(h1, w2_ref[...], preferred_element_type=jnp.float32)


def _tc2_body(y_ref, s_ref, d_ref, h1_ref, wr_ref, b2_ref, w3_ref, b3_ref,
              w4_ref, b4_ref, w5_ref, b5_ref, o_ref):
    yb = y_ref[...]
    h2 = _lrelu((s_ref[0] + s_ref[1] + yb) * _dv_block(d_ref) + b2_ref[...]
                + jnp.dot(h1_ref[...], wr_ref[...],
                          preferred_element_type=jnp.float32))
    h3 = _lrelu(jnp.dot(h2, w3_ref[...],
                        preferred_element_type=jnp.float32) + b3_ref[...])
    h4 = _lrelu(jnp.dot(h3, w4_ref[...],
                        preferred_element_type=jnp.float32) + b4_ref[...])
    o_ref[...] = jnp.dot(h4, w5_ref[...],
                         preferred_element_type=jnp.float32) + b5_ref[...]


def _full(shape):
    return pl.BlockSpec(shape, lambda i: tuple(0 for _ in shape))


def _tc1(xp, s1, dp, w1_out, b1, w1_root, w2_out):
    return pl.pallas_call(
        _tc1_body,
        grid=(_NP // _RP,),
        in_specs=[
            pl.BlockSpec((_RP, _F), lambda i: (i, 0)),
            pl.BlockSpec((_NSC, _RP, _F), lambda i: (0, i, 0)),
            pl.BlockSpec((_NSC * _NTILES, _RP // _F, _F),
                         lambda i: (0, i, 0)),
            _full((_F, 256)),
            _full((1, 256)),
            _full((_F, 256)),
            _full((256, _F)),
        ],
        out_specs=[
            pl.BlockSpec((_RP, 256), lambda i: (i, 0)),
            pl.BlockSpec((_RP, _F), lambda i: (i, 0)),
        ],
        out_shape=[
            jax.ShapeDtypeStruct((_NP, 256), jnp.float32),
            jax.ShapeDtypeStruct((_NP, _F), jnp.float32),
        ],
    )(xp, s1, dp, w1_out, b1, w1_root, w2_out)


def _tc2(y2, s2, dp, h1, w2_root, b2, w3, b3, w4, b4, w5, b5):
    return pl.pallas_call(
        _tc2_body,
        grid=(_NP // _RP,),
        in_specs=[
            pl.BlockSpec((_RP, _F), lambda i: (i, 0)),
            pl.BlockSpec((_NSC, _RP, _F), lambda i: (0, i, 0)),
            pl.BlockSpec((_NSC * _NTILES, _RP // _F, _F),
                         lambda i: (0, i, 0)),
            pl.BlockSpec((_RP, 256), lambda i: (i, 0)),
            _full((256, _F)),
            _full((1, _F)),
            _full((_F, 256)),
            _full((1, 256)),
            _full((256, _F)),
            _full((1, _F)),
            _full((_F, 64)),
            _full((1, 64)),
        ],
        out_specs=pl.BlockSpec((_RP, 64), lambda i: (i, 0)),
        out_shape=jax.ShapeDtypeStruct((_NP, 64), jnp.float32),
    )(y2, s2, dp, h1, w2_root, b2, w3, b3, w4, b4, w5, b5)


def kernel(x, edge_index, w1_out, b1_out, w1_root, w2_out, b2_out, w2_root,
           w3, b3, w4, b4, w5, b5):
    nwin_total = _EPAD // _W + 1
    pad = (jnp.arange(_E, _EPAD + _W, dtype=jnp.int32) % _N)
    rowp = jnp.concatenate([edge_index[0], pad]).reshape(nwin_total, 1, _W)
    colp = jnp.concatenate([edge_index[1], pad]).reshape(nwin_total, 1, _W)
    idxarr = jnp.concatenate([rowp, colp], axis=1)
    zf = jnp.zeros((_NP, _F), jnp.float32)
    zh = jnp.zeros((_HR, _F), jnp.int32)

    xp = jnp.pad(x, ((0, _NP - _N), (0, 0)))
    s1, dp = _sc_pass(True)(xp, idxarr, zf, zh)
    h1, y2 = _tc1(xp, s1, dp, w1_out, b1_out.reshape(1, -1), w1_root, w2_out)
    (s2,) = _sc_pass(False)(y2, idxarr, zf)
    out = _tc2(y2, s2, dp, h1, w2_root, b2_out.reshape(1, -1),
               w3, b3.reshape(1, -1), w4, b4.reshape(1, -1),
               w5, b5.reshape(1, -1))
    return out[:_N]


# speedup vs baseline: 1.0338x; 1.0175x over previous
import jax
import jax.numpy as jnp
from jax import lax
from jax.experimental import pallas as pl
from jax.experimental.pallas import tpu as pltpu
from jax.experimental.pallas import tpu_sc as plsc

_N = 10000
_F = 128
_E = 320000
_NP = 10240
_W = 128
_LANES = 16
_NTILES = 16
_NSC = 2
_EPAD = 327680
_NWIN = _EPAD // (_NSC * _NTILES * _W)
_HR = _NP // _F


def _sc_pass(with_deg):
    mesh = plsc.VectorSubcoreMesh(core_axis_name="c", subcore_axis_name="s")
    out_type = [jax.ShapeDtypeStruct((_NSC, _NP, _F), jnp.float32)]
    scratch = [
        pltpu.VMEM((2, _W), jnp.int32),
        pltpu.VMEM((2, _W), jnp.int32),
        pltpu.VMEM((_W,), jnp.int32),
        pltpu.VMEM((_W,), jnp.int32),
        pltpu.VMEM((_W, _F), jnp.float32),
        pltpu.VMEM((_W, _F), jnp.float32),
        pltpu.VMEM_SHARED((_NP, _F), jnp.float32),
        pltpu.SemaphoreType.DMA,
        pltpu.SemaphoreType.DMA,
        pltpu.SemaphoreType.DMA,
        pltpu.SemaphoreType.DMA,
        pltpu.SemaphoreType.DMA,
        pltpu.SemaphoreType.DMA,
    ]
    if with_deg:
        out_type.append(
            jax.ShapeDtypeStruct((_NSC * _NTILES, _HR, _F), jnp.int32))
        scratch.append(pltpu.VMEM((_HR, _F), jnp.int32))

    def body(*args):
        if with_deg:
            (v_hbm, idx_hbm, zf_hbm, zh_hbm, s_out, d_out,
             idx0, idx1, colp0, colp1, xb0, xb1, agg,
             isem0, isem1, gsem0, gsem1, ssem0, ssem1, hist_v) = args
        else:
            (v_hbm, idx_hbm, zf_hbm, s_out,
             idx0, idx1, colp0, colp1, xb0, xb1, agg,
             isem0, isem1, gsem0, gsem1, ssem0, ssem1) = args
        idx = (idx0, idx1)
        colp = (colp0, colp1)
        xb = (xb0, xb1)
        isem = (isem0, isem1)
        gsem = (gsem0, gsem1)
        ssem = (ssem0, ssem1)
        c = lax.axis_index("c")
        s = lax.axis_index("s")
        zrows = _NP // _NTILES
        pltpu.sync_copy(zf_hbm, agg.at[pl.ds(s * zrows, zrows)])
        if with_deg:
            pltpu.sync_copy(zh_hbm, hist_v)
        plsc.subcore_barrier()

        lane = lax.iota(jnp.int32, 16)
        wbase = (c * _NTILES + s) * _NWIN

        def compute_colp(b):
            rv, cv = idx[b].at[0], idx[b].at[1]
            for i in range(_W // 16):
                r = rv[pl.ds(i * 16, 16)]
                cc = cv[pl.ds(i * 16, 16)]
                colp[b][pl.ds(i * 16, 16)] = jnp.where(r == cc, _N + lane, cc)

        def deg_update(b):
            rv, cv = idx[b].at[0], idx[b].at[1]
            for i in range(_W // 16):
                r = rv[pl.ds(i * 16, 16)]
                cc = cv[pl.ds(i * 16, 16)]
                cnt, last = plsc.scan_count(cc, r != cc)
                plsc.addupdate_scatter(
                    hist_v, [cc >> 7, cc & 127], cnt, mask=last)

        def start_gather(b):
            pltpu.async_copy(v_hbm.at[idx[b].at[0]], xb[b], gsem[b])

        def wait_gather(b):
            pltpu.make_async_copy(v_hbm.at[idx[b].at[0]], xb[b],
                                  gsem[b]).wait()

        def start_scatter(b):
            pltpu.async_copy(xb[b], agg.at[colp[b]], ssem[b], add=True)

        def wait_scatter(b):
            pltpu.make_async_copy(xb[b], agg.at[colp[b]], ssem[b]).wait()

        pltpu.sync_copy(idx_hbm.at[wbase], idx0)
        compute_colp(0)
        start_gather(0)

        def pair(k, carry):
            for b in (0, 1):
                g = 2 * k + b
                o = 1 - b
                pltpu.async_copy(idx_hbm.at[wbase + g + 1], idx[o], isem[o])
                if with_deg:
                    deg_update(b)
                wait_gather(b)
                start_scatter(b)
                if b == 0:

                    @pl.when(k > 0)
                    def _():
                        wait_scatter(o)
                else:
                    wait_scatter(o)
                pltpu.make_async_copy(idx_hbm.at[wbase + g + 1], idx[o],
                                      isem[o]).wait()
                compute_colp(o)
                start_gather(o)
            return carry

        lax.fori_loop(0, _NWIN // 2, pair, 0)
        wait_scatter(1)
        wait_gather(0)
        plsc.subcore_barrier()
        orows = _NP // _NTILES
        pltpu.sync_copy(agg.at[pl.ds(s * orows, orows)],
                        s_out.at[c, pl.ds(s * orows, orows)])
        if with_deg:
            pltpu.sync_copy(hist_v, d_out.at[c * _NTILES + s])

    params = (pltpu.CompilerParams(needs_layout_passes=False)
              if with_deg else None)
    cost = pl.CostEstimate(flops=0, transcendentals=0,
                           bytes_accessed=340_000_000)
    return pl.kernel(body, out_type=out_type, mesh=mesh,
                     scratch_types=scratch, compiler_params=params,
                     cost_estimate=cost)


_RP = 1024


def _lrelu(h):
    return jnp.where(h >= 0, h, 0.01 * h)


def _dv_block(d_ref):
    deg = jnp.sum(d_ref[...], axis=0).astype(jnp.float32) + 1.0
    dinv = 1.0 / deg
    ey = (lax.broadcasted_iota(jnp.int32, (_F, _F), 0)
          == lax.broadcasted_iota(jnp.int32, (_F, _F), 1)).astype(jnp.float32)
    ones = jnp.ones((_F, _F), jnp.float32)
    rows = [jnp.dot(ey * dinv[r:r + 1, :], ones,
                    preferred_element_type=jnp.float32)
            for r in range(_RP // _F)]
    return jnp.concatenate(rows, axis=0)


def _tc1_body(x_ref, s_ref, d_ref, wo_ref, b_ref, wr_ref, w2_ref,
              h1_ref, y2_ref):
    xb = x_ref[...]
    agg = (s_ref[0] + s_ref[1] + xb) * _dv_block(d_ref)
    h = (jnp.dot(agg, wo_ref[...], preferred_element_type=jnp.float32)
         + b_ref[...]
         + jnp.dot(xb, wr_ref[...], preferred_element_type=jnp.float32))
    h1 = _lrelu(h)
    h1_ref[...] = h1
    y2_ref[...] = jnp.dot(h1, w2_ref[...], preferred_element_type=jnp.float32)


def _tc2_body(y_ref, s_ref, d_ref, h1_ref, wr_ref, b2_ref, w3_ref, b3_ref,
              w4_ref, b4_ref, w5_ref, b5_ref, o_ref):
    yb = y_ref[...]
    h2 = _lrelu((s_ref[0] + s_ref[1] + yb) * _dv_block(d_ref) + b2_ref[...]
                + jnp.dot(h1_ref[...], wr_ref[...],
                          preferred_element_type=jnp.float32))
    h3 = _lrelu(jnp.dot(h2, w3_ref[...],
                        preferred_element_type=jnp.float32) + b3_ref[...])
    h4 = _lrelu(jnp.dot(h3, w4_ref[...],
                        preferred_element_type=jnp.float32) + b4_ref[...])
    o_ref[...] = jnp.dot(h4, w5_ref[...],
                         preferred_element_type=jnp.float32) + b5_ref[...]


def _full(shape):
    return pl.BlockSpec(shape, lambda i: tuple(0 for _ in shape))


def _tc1(xp, s1, dp, w1_out, b1, w1_root, w2_out):
    return pl.pallas_call(
        _tc1_body,
        grid=(_NP // _RP,),
        in_specs=[
            pl.BlockSpec((_RP, _F), lambda i: (i, 0)),
            pl.BlockSpec((_NSC, _RP, _F), lambda i: (0, i, 0)),
            pl.BlockSpec((_NSC * _NTILES, _RP // _F, _F),
                         lambda i: (0, i, 0)),
            _full((_F, 256)),
            _full((1, 256)),
            _full((_F, 256)),
            _full((256, _F)),
        ],
        out_specs=[
            pl.BlockSpec((_RP, 256), lambda i: (i, 0)),
            pl.BlockSpec((_RP, _F), lambda i: (i, 0)),
        ],
        out_shape=[
            jax.ShapeDtypeStruct((_NP, 256), jnp.float32),
            jax.ShapeDtypeStruct((_NP, _F), jnp.float32),
        ],
    )(xp, s1, dp, w1_out, b1, w1_root, w2_out)


def _tc2(y2, s2, dp, h1, w2_root, b2, w3, b3, w4, b4, w5, b5):
    return pl.pallas_call(
        _tc2_body,
        grid=(_NP // _RP,),
        in_specs=[
            pl.BlockSpec((_RP, _F), lambda i: (i, 0)),
            pl.BlockSpec((_NSC, _RP, _F), lambda i: (0, i, 0)),
            pl.BlockSpec((_NSC * _NTILES, _RP // _F, _F),
                         lambda i: (0, i, 0)),
            pl.BlockSpec((_RP, 256), lambda i: (i, 0)),
            _full((256, _F)),
            _full((1, _F)),
            _full((_F, 256)),
            _full((1, 256)),
            _full((256, _F)),
            _full((1, _F)),
            _full((_F, 64)),
            _full((1, 64)),
        ],
        out_specs=pl.BlockSpec((_RP, 64), lambda i: (i, 0)),
        out_shape=jax.ShapeDtypeStruct((_NP, 64), jnp.float32),
    )(y2, s2, dp, h1, w2_root, b2, w3, b3, w4, b4, w5, b5)


def kernel(x, edge_index, w1_out, b1_out, w1_root, w2_out, b2_out, w2_root,
           w3, b3, w4, b4, w5, b5):
    nwin_total = _EPAD // _W + 1
    pad = (jnp.arange(_E, _EPAD + _W, dtype=jnp.int32) % _N)
    rowp = jnp.concatenate([edge_index[0], pad]).reshape(nwin_total, 1, _W)
    colp = jnp.concatenate([edge_index[1], pad]).reshape(nwin_total, 1, _W)
    idxarr = jnp.concatenate([rowp, colp], axis=1)
    zf = jnp.zeros((_NP // _NTILES, _F), jnp.float32)
    zh = jnp.zeros((_HR, _F), jnp.int32)

    s1, dp = _sc_pass(True)(x, idxarr, zf, zh)
    h1, y2 = _tc1(x, s1, dp, w1_out, b1_out.reshape(1, -1), w1_root, w2_out)
    (s2,) = _sc_pass(False)(y2, idxarr, zf)
    out = _tc2(y2, s2, dp, h1, w2_root, b2_out.reshape(1, -1),
               w3, b3.reshape(1, -1), w4, b4.reshape(1, -1),
               w5, b5.reshape(1, -1))
    return out[:_N]


# speedup vs baseline: 1.0363x; 1.0024x over previous
import jax
import jax.numpy as jnp
from jax import lax
from jax.experimental import pallas as pl
from jax.experimental.pallas import tpu as pltpu
from jax.experimental.pallas import tpu_sc as plsc

_N = 10000
_F = 128
_E = 320000
_NP = 10240
_W = 128
_LANES = 16
_NTILES = 16
_NSC = 2
_EPAD = 327680
_NWIN = _EPAD // (_NSC * _NTILES * _W)
_HR = _NP // _F


def _sc_pass(with_deg):
    mesh = plsc.VectorSubcoreMesh(core_axis_name="c", subcore_axis_name="s")
    out_type = [jax.ShapeDtypeStruct((_NSC, _NP, _F), jnp.float32)]
    scratch = [
        pltpu.VMEM((2, _W), jnp.int32),
        pltpu.VMEM((2, _W), jnp.int32),
        pltpu.VMEM((_W,), jnp.int32),
        pltpu.VMEM((_W,), jnp.int32),
        pltpu.VMEM((_W, _F), jnp.float32),
        pltpu.VMEM((_W, _F), jnp.float32),
        pltpu.VMEM_SHARED((_NP, _F), jnp.float32),
        pltpu.SemaphoreType.DMA,
        pltpu.SemaphoreType.DMA,
        pltpu.SemaphoreType.DMA,
        pltpu.SemaphoreType.DMA,
        pltpu.SemaphoreType.DMA,
        pltpu.SemaphoreType.DMA,
    ]
    if with_deg:
        out_type.append(
            jax.ShapeDtypeStruct((_NSC * _NTILES, _HR, _F), jnp.int32))
        scratch.append(pltpu.VMEM((_HR, _F), jnp.int32))

    def body(*args):
        if with_deg:
            (v_hbm, idx_hbm, zf_hbm, zh_hbm, s_out, d_out,
             idx0, idx1, colp0, colp1, xb0, xb1, agg,
             isem0, isem1, gsem0, gsem1, ssem0, ssem1, hist_v) = args
        else:
            (v_hbm, idx_hbm, zf_hbm, s_out,
             idx0, idx1, colp0, colp1, xb0, xb1, agg,
             isem0, isem1, gsem0, gsem1, ssem0, ssem1) = args
        idx = (idx0, idx1)
        colp = (colp0, colp1)
        xb = (xb0, xb1)
        isem = (isem0, isem1)
        gsem = (gsem0, gsem1)
        ssem = (ssem0, ssem1)
        c = lax.axis_index("c")
        s = lax.axis_index("s")
        zrows = _NP // _NTILES
        pltpu.sync_copy(zf_hbm, agg.at[pl.ds(s * zrows, zrows)])
        if with_deg:
            pltpu.sync_copy(zh_hbm, hist_v)
        plsc.subcore_barrier()

        lane = lax.iota(jnp.int32, 16)
        wbase = (c * _NTILES + s) * _NWIN

        def compute_colp(b):
            rv, cv = idx[b].at[0], idx[b].at[1]
            for i in range(_W // 16):
                r = rv[pl.ds(i * 16, 16)]
                cc = cv[pl.ds(i * 16, 16)]
                colp[b][pl.ds(i * 16, 16)] = jnp.where(r == cc, _N + lane, cc)

        def deg_update(b):
            rv, cv = idx[b].at[0], idx[b].at[1]
            for i in range(_W // 16):
                r = rv[pl.ds(i * 16, 16)]
                cc = cv[pl.ds(i * 16, 16)]
                cnt, last = plsc.scan_count(cc, r != cc)
                plsc.addupdate_scatter(
                    hist_v, [cc >> 7, cc & 127], cnt, mask=last)

        def start_gather(b):
            pltpu.async_copy(v_hbm.at[idx[b].at[0]], xb[b], gsem[b])

        def wait_gather(b):
            pltpu.make_async_copy(v_hbm.at[idx[b].at[0]], xb[b],
                                  gsem[b]).wait()

        def start_scatter(b):
            pltpu.async_copy(xb[b], agg.at[colp[b]], ssem[b], add=True)

        def wait_scatter(b):
            pltpu.make_async_copy(xb[b], agg.at[colp[b]], ssem[b]).wait()

        pltpu.sync_copy(idx_hbm.at[wbase], idx0)
        compute_colp(0)
        start_gather(0)
        pltpu.async_copy(idx_hbm.at[wbase + 1], idx1, isem1)

        def pair(k, carry):
            for b in (0, 1):
                g = 2 * k + b
                o = 1 - b
                if with_deg:
                    deg_update(b)
                wait_gather(b)
                start_scatter(b)
                if b == 0:

                    @pl.when(k > 0)
                    def _():
                        wait_scatter(o)
                else:
                    wait_scatter(o)
                pltpu.make_async_copy(idx_hbm.at[wbase + g + 1], idx[o],
                                      isem[o]).wait()
                compute_colp(o)
                start_gather(o)
                pltpu.async_copy(idx_hbm.at[wbase + g + 2], idx[b], isem[b])
            return carry

        lax.fori_loop(0, _NWIN // 2, pair, 0)
        wait_scatter(1)
        wait_gather(0)
        pltpu.make_async_copy(idx_hbm.at[wbase + _NWIN + 1], idx[1],
                              isem[1]).wait()
        plsc.subcore_barrier()
        orows = _NP // _NTILES
        pltpu.sync_copy(agg.at[pl.ds(s * orows, orows)],
                        s_out.at[c, pl.ds(s * orows, orows)])
        if with_deg:
            pltpu.sync_copy(hist_v, d_out.at[c * _NTILES + s])

    params = (pltpu.CompilerParams(needs_layout_passes=False)
              if with_deg else None)
    cost = pl.CostEstimate(flops=0, transcendentals=0,
                           bytes_accessed=340_000_000)
    return pl.kernel(body, out_type=out_type, mesh=mesh,
                     scratch_types=scratch, compiler_params=params,
                     cost_estimate=cost)


_RP = 1024


def _lrelu(h):
    return jnp.where(h >= 0, h, 0.01 * h)


def _dv_block(d_ref):
    deg = jnp.sum(d_ref[...], axis=0).astype(jnp.float32) + 1.0
    dinv = 1.0 / deg
    ey = (lax.broadcasted_iota(jnp.int32, (_F, _F), 0)
          == lax.broadcasted_iota(jnp.int32, (_F, _F), 1)).astype(jnp.float32)
    ones = jnp.ones((_F, _F), jnp.float32)
    rows = [jnp.dot(ey * dinv[r:r + 1, :], ones,
                    preferred_element_type=jnp.float32)
            for r in range(_RP // _F)]
    return jnp.concatenate(rows, axis=0)


def _tc1_body(x_ref, s_ref, d_ref, wo_ref, b_ref, wr_ref, w2_ref,
              h1_ref, y2_ref):
    xb = x_ref[...]
    agg = (s_ref[0] + s_ref[1] + xb) * _dv_block(d_ref)
    h = (jnp.dot(agg, wo_ref[...], preferred_element_type=jnp.float32)
         + b_ref[...]
         + jnp.dot(xb, wr_ref[...], preferred_element_type=jnp.float32))
    h1 = _lrelu(h)
    h1_ref[...] = h1
    y2_ref[...] = jnp.dot(h1, w2_ref[...], preferred_element_type=jnp.float32)


def _tc2_body(y_ref, s_ref, d_ref, h1_ref, wr_ref, b2_ref, w3_ref, b3_ref,
              w4_ref, b4_ref, w5_ref, b5_ref, o_ref):
    yb = y_ref[...]
    h2 = _lrelu((s_ref[0] + s_ref[1] + yb) * _dv_block(d_ref) + b2_ref[...]
                + jnp.dot(h1_ref[...], wr_ref[...],
                          preferred_element_type=jnp.float32))
    h3 = _lrelu(jnp.dot(h2, w3_ref[...],
                        preferred_element_type=jnp.float32) + b3_ref[...])
    h4 = _lrelu(jnp.dot(h3, w4_ref[...],
                        preferred_element_type=jnp.float32) + b4_ref[...])
    o_ref[...] = jnp.dot(h4, w5_ref[...],
                         preferred_element_type=jnp.float32) + b5_ref[...]


def _full(shape):
    return pl.BlockSpec(shape, lambda i: tuple(0 for _ in shape))


def _tc1(xp, s1, dp, w1_out, b1, w1_root, w2_out):
    return pl.pallas_call(
        _tc1_body,
        grid=(_NP // _RP,),
        in_specs=[
            pl.BlockSpec((_RP, _F), lambda i: (i, 0)),
            pl.BlockSpec((_NSC, _RP, _F), lambda i: (0, i, 0)),
            pl.BlockSpec((_NSC * _NTILES, _RP // _F, _F),
                         lambda i: (0, i, 0)),
            _full((_F, 256)),
            _full((1, 256)),
            _full((_F, 256)),
            _full((256, _F)),
        ],
        out_specs=[
            pl.BlockSpec((_RP, 256), lambda i: (i, 0)),
            pl.BlockSpec((_RP, _F), lambda i: (i, 0)),
        ],
        out_shape=[
            jax.ShapeDtypeStruct((_NP, 256), jnp.float32),
            jax.ShapeDtypeStruct((_NP, _F), jnp.float32),
        ],
    )(xp, s1, dp, w1_out, b1, w1_root, w2_out)


def _tc2(y2, s2, dp, h1, w2_root, b2, w3, b3, w4, b4, w5, b5):
    return pl.pallas_call(
        _tc2_body,
        grid=(_NP // _RP,),
        in_specs=[
            pl.BlockSpec((_RP, _F), lambda i: (i, 0)),
            pl.BlockSpec((_NSC, _RP, _F), lambda i: (0, i, 0)),
            pl.BlockSpec((_NSC * _NTILES, _RP // _F, _F),
                         lambda i: (0, i, 0)),
            pl.BlockSpec((_RP, 256), lambda i: (i, 0)),
            _full((256, _F)),
            _full((1, _F)),
            _full((_F, 256)),
            _full((1, 256)),
            _full((256, _F)),
            _full((1, _F)),
            _full((_F, 64)),
            _full((1, 64)),
        ],
        out_specs=pl.BlockSpec((_RP, 64), lambda i: (i, 0)),
        out_shape=jax.ShapeDtypeStruct((_NP, 64), jnp.float32),
    )(y2, s2, dp, h1, w2_root, b2, w3, b3, w4, b4, w5, b5)


def kernel(x, edge_index, w1_out, b1_out, w1_root, w2_out, b2_out, w2_root,
           w3, b3, w4, b4, w5, b5):
    nwin_total = _EPAD // _W + 1
    pad = (jnp.arange(_E, _EPAD + _W, dtype=jnp.int32) % _N)
    rowp = jnp.concatenate([edge_index[0], pad]).reshape(nwin_total, 1, _W)
    colp = jnp.concatenate([edge_index[1], pad]).reshape(nwin_total, 1, _W)
    idxarr = jnp.concatenate([rowp, colp], axis=1)
    zf = jnp.zeros((_NP // _NTILES, _F), jnp.float32)
    zh = jnp.zeros((_HR, _F), jnp.int32)

    s1, dp = _sc_pass(True)(x, idxarr, zf, zh)
    h1, y2 = _tc1(x, s1, dp, w1_out, b1_out.reshape(1, -1), w1_root, w2_out)
    (s2,) = _sc_pass(False)(y2, idxarr, zf)
    out = _tc2(y2, s2, dp, h1, w2_root, b2_out.reshape(1, -1),
               w3, b3.reshape(1, -1), w4, b4.reshape(1, -1),
               w5, b5.reshape(1, -1))
    return out[:_N]
